# Initial kernel scaffold; baseline (speedup 1.0000x reference)
import jax
import jax.numpy as jnp
from jax import lax
from jax.experimental import pallas as pl
from jax.experimental.pallas import tpu as pltpu
from jax.experimental.pallas import tpu_sc as plsc

_NC, _NS = 2, 16
_NW = _NC * _NS
_CHUNK = 128
_LANES = 16


def _round_up(a: int, b: int) -> int:
    return (a + b - 1) // b * b


def _make_sc_scatter(n_pad: int, n_chunks: int, d: int, with_count: bool):
    rows_pt = n_pad // _NS

    def body(table, srcr, dstr, *rest):
        if with_count:
            (z2, z1, acc_out, cnt_out,
             src_v, dst_v, rows_v, ones_v, acc_sh, cnt_sh, gsem) = rest
        else:
            (z2, acc_out,
             src_v, dst_v, rows_v, acc_sh, gsem) = rest
        c = lax.axis_index("c")
        s = lax.axis_index("s")
        row0 = s * rows_pt
        rows = pl.ds(row0, rows_pt)
        pltpu.sync_copy(z2.at[rows], acc_sh.at[rows])
        if with_count:
            pltpu.sync_copy(z1.at[rows], cnt_sh.at[rows])
            for j in range(_CHUNK // _LANES):
                ones_v[pl.ds(j * _LANES, _LANES)] = jnp.ones(
                    (_LANES,), jnp.float32)
        pltpu.sync_copy(srcr.at[c, s], src_v)
        pltpu.sync_copy(dstr.at[c, s], dst_v)
        plsc.subcore_barrier()

        def chunk(i, carry):
            pltpu.async_copy(table.at[src_v.at[i]], rows_v, gsem).wait()
            pltpu.sync_copy(rows_v, acc_sh.at[dst_v.at[i]], add=True)
            if with_count:
                pltpu.sync_copy(ones_v, cnt_sh.at[dst_v.at[i]], add=True)
            return carry

        lax.fori_loop(0, n_chunks, chunk, 0)
        plsc.subcore_barrier()
        pltpu.sync_copy(acc_sh.at[rows], acc_out.at[c, rows])
        if with_count:
            pltpu.sync_copy(cnt_sh.at[rows], cnt_out.at[c, rows])

    out_type = [jax.ShapeDtypeStruct((_NC, n_pad, d), jnp.float32)]
    scratch = [
        pltpu.VMEM((n_chunks, _CHUNK), jnp.int32),
        pltpu.VMEM((n_chunks, _CHUNK), jnp.int32),
        pltpu.VMEM((_CHUNK, d), jnp.float32),
    ]
    if with_count:
        out_type.append(jax.ShapeDtypeStruct((_NC, n_pad), jnp.float32))
        scratch.append(pltpu.VMEM((_CHUNK,), jnp.float32))
    scratch.append(pltpu.VMEM_SHARED((n_pad, d), jnp.float32))
    if with_count:
        scratch.append(pltpu.VMEM_SHARED((n_pad,), jnp.float32))
    scratch.append(pltpu.SemaphoreType.DMA)

    mesh = plsc.VectorSubcoreMesh(core_axis_name="c", subcore_axis_name="s",
                                  num_cores=_NC, num_subcores=_NS)
    return pl.kernel(body, out_type=out_type, mesh=mesh,
                     scratch_types=scratch)


def _matmul_body(x_ref, w_ref, o_ref):
    o_ref[...] = jnp.dot(x_ref[...], w_ref[...],
                         preferred_element_type=jnp.float32)


def _mid_body(p0, p1, c0, c1, b, w, o):
    cnt = jnp.maximum(c0[...] + c1[...], 1.0)
    h = (p0[...] + p1[...]) / cnt + b[...]
    h = jnp.where(h >= 0.0, h, 0.01 * h)
    o[...] = jnp.dot(h, w[...], preferred_element_type=jnp.float32)


def _fin_body(p0, p1, c0, c1, b, o):
    cnt = jnp.maximum(c0[...] + c1[...], 1.0)
    o[...] = (p0[...] + p1[...]) / cnt + b[...]


def _tc_call(body, n, d, bn, w_shape=None, out_d=None):
    out_d = out_d or d
    grid = (n // bn,)
    in_specs = [pl.BlockSpec((bn, d), lambda i: (i, 0)),
                pl.BlockSpec((bn, d), lambda i: (i, 0)),
                pl.BlockSpec((bn, 1), lambda i: (i, 0)),
                pl.BlockSpec((bn, 1), lambda i: (i, 0)),
                pl.BlockSpec((1, out_d), lambda i: (0, 0))]
    if w_shape is not None:
        in_specs.append(pl.BlockSpec(w_shape, lambda i: (0, 0)))
    return pl.pallas_call(
        body, grid=grid, in_specs=in_specs,
        out_specs=pl.BlockSpec((bn, out_d), lambda i: (i, 0)),
        out_shape=jax.ShapeDtypeStruct((n, out_d), jnp.float32))


def kernel(x, edge_index, W1, b1, W2, b2):
    n, d_in = x.shape
    d_h = W1.shape[0]
    d_out = W2.shape[0]
    e = edge_index.shape[1]

    src = edge_index[0].astype(jnp.int32)
    dst = edge_index[1].astype(jnp.int32)
    per_tile = _round_up(e, _NW * _CHUNK) // _NW
    n_chunks = per_tile // _CHUNK
    e_pad = per_tile * _NW
    if e_pad > e:
        src = jnp.concatenate([src, jnp.zeros((e_pad - e,), jnp.int32)])
        dst = jnp.concatenate([dst, jnp.full((e_pad - e,), n, jnp.int32)])
    srcr = src.reshape(_NC, _NS, n_chunks, _CHUNK)
    dstr = dst.reshape(_NC, _NS, n_chunks, _CHUNK)

    n_pad = _round_up(n + 1, _NS * 8)
    z2 = jnp.zeros((n_pad, d_h), jnp.float32)
    z1 = jnp.zeros((n_pad,), jnp.float32)

    bn = 2000
    mm1 = pl.pallas_call(
        _matmul_body, grid=(n // bn,),
        in_specs=[pl.BlockSpec((bn, d_in), lambda i: (i, 0)),
                  pl.BlockSpec((d_in, d_h), lambda i: (0, 0))],
        out_specs=pl.BlockSpec((bn, d_h), lambda i: (i, 0)),
        out_shape=jax.ShapeDtypeStruct((n, d_h), jnp.float32))
    g1 = mm1(x, W1.T)

    scat1 = _make_sc_scatter(n_pad, n_chunks, d_h, with_count=True)
    parts1, counts = scat1(g1, srcr, dstr, z2, z1)

    p0 = parts1[0, :n]
    p1 = parts1[1, :n]
    c0 = counts[0, :n, None]
    c1 = counts[1, :n, None]

    mid = _tc_call(_mid_body, n, d_h, bn,
                   w_shape=(d_h, d_out), out_d=d_out)
    g2 = mid(p0, p1, c0, c1, b1.reshape(1, d_h), W2.T)

    scat2 = _make_sc_scatter(n_pad, n_chunks, d_out, with_count=False)
    (parts2,) = scat2(g2, srcr, dstr, z2)

    q0 = parts2[0, :n]
    q1 = parts2[1, :n]
    fin = _tc_call(_fin_body, n, d_out, bn)
    out = fin(q0, q1, c0, c1, b2.reshape(1, d_out))
    return out


# speedup vs baseline: 5.0965x; 5.0965x over previous
import jax
import jax.numpy as jnp
from jax import lax
from jax.experimental import pallas as pl
from jax.experimental.pallas import tpu as pltpu
from jax.experimental.pallas import tpu_sc as plsc

_NC, _NS = 2, 16
_NW = _NC * _NS
_CHUNK = 128
_LANES = 16


def _round_up(a: int, b: int) -> int:
    return (a + b - 1) // b * b


def _make_sc_scatter(n_pad: int, n_chunks: int, d: int, with_count: bool):
    rows_pt = n_pad // _NS

    def body(table, srcr, dstr, *rest):
        if with_count:
            (z2, acc_out, cnt_out,
             src_v, dst_v, rows_v, ones_v, cbuf, acc_sh, cnt_sh, gsem) = rest
        else:
            (z2, acc_out,
             src_v, dst_v, rows_v, acc_sh, gsem) = rest
        c = lax.axis_index("c")
        s = lax.axis_index("s")
        row0 = s * rows_pt
        rows = pl.ds(row0, rows_pt)
        pltpu.sync_copy(z2.at[rows], acc_sh.at[rows])
        if with_count:
            for j in range(rows_pt // _LANES):
                cbuf[pl.ds(j * _LANES, _LANES)] = jnp.zeros(
                    (_LANES,), jnp.float32)
            pltpu.sync_copy(cbuf, cnt_sh.at[rows])
            for j in range(_CHUNK // _LANES):
                ones_v[pl.ds(j * _LANES, _LANES)] = jnp.ones(
                    (_LANES,), jnp.float32)
        pltpu.sync_copy(srcr.at[c, s], src_v)
        pltpu.sync_copy(dstr.at[c, s], dst_v)
        plsc.subcore_barrier()

        def chunk(i, carry):
            pltpu.async_copy(table.at[src_v.at[i]], rows_v, gsem).wait()
            pltpu.sync_copy(rows_v, acc_sh.at[dst_v.at[i]], add=True)
            if with_count:
                pltpu.sync_copy(ones_v, cnt_sh.at[dst_v.at[i]], add=True)
            return carry

        lax.fori_loop(0, n_chunks, chunk, 0)
        plsc.subcore_barrier()
        pltpu.sync_copy(acc_sh.at[rows], acc_out.at[c, rows])
        if with_count:
            pltpu.sync_copy(cnt_sh.at[rows], cbuf)
            pltpu.sync_copy(cbuf,
                            cnt_out.at[pl.ds(c * n_pad + row0, rows_pt)])

    out_type = [jax.ShapeDtypeStruct((_NC, n_pad, d), jnp.float32)]
    scratch = [
        pltpu.VMEM((n_chunks, _CHUNK), jnp.int32),
        pltpu.VMEM((n_chunks, _CHUNK), jnp.int32),
        pltpu.VMEM((_CHUNK, d), jnp.float32),
    ]
    if with_count:
        out_type.append(jax.ShapeDtypeStruct((_NC * n_pad,), jnp.float32))
        scratch.append(pltpu.VMEM((_CHUNK,), jnp.float32))
        scratch.append(pltpu.VMEM((rows_pt,), jnp.float32))
    scratch.append(pltpu.VMEM_SHARED((n_pad, d), jnp.float32))
    if with_count:
        scratch.append(pltpu.VMEM_SHARED((n_pad,), jnp.float32))
    scratch.append(pltpu.SemaphoreType.DMA)

    mesh = plsc.VectorSubcoreMesh(core_axis_name="c", subcore_axis_name="s",
                                  num_cores=_NC, num_subcores=_NS)
    return pl.kernel(body, out_type=out_type, mesh=mesh,
                     scratch_types=scratch)


def _matmul_body(x_ref, w_ref, o_ref):
    o_ref[...] = jnp.dot(x_ref[...], w_ref[...],
                         preferred_element_type=jnp.float32)


def _mid_body(p0, p1, c0, c1, b, w, o):
    cnt = jnp.maximum(c0[...] + c1[...], 1.0)
    h = (p0[...] + p1[...]) / cnt + b[...]
    h = jnp.where(h >= 0.0, h, 0.01 * h)
    o[...] = jnp.dot(h, w[...], preferred_element_type=jnp.float32)


def _fin_body(p0, p1, c0, c1, b, o):
    cnt = jnp.maximum(c0[...] + c1[...], 1.0)
    o[...] = (p0[...] + p1[...]) / cnt + b[...]


def _tc_call(body, n, d, bn, w_shape=None, out_d=None):
    out_d = out_d or d
    grid = (n // bn,)
    in_specs = [pl.BlockSpec((bn, d), lambda i: (i, 0)),
                pl.BlockSpec((bn, d), lambda i: (i, 0)),
                pl.BlockSpec((bn, 1), lambda i: (i, 0)),
                pl.BlockSpec((bn, 1), lambda i: (i, 0)),
                pl.BlockSpec((1, out_d), lambda i: (0, 0))]
    if w_shape is not None:
        in_specs.append(pl.BlockSpec(w_shape, lambda i: (0, 0)))
    return pl.pallas_call(
        body, grid=grid, in_specs=in_specs,
        out_specs=pl.BlockSpec((bn, out_d), lambda i: (i, 0)),
        out_shape=jax.ShapeDtypeStruct((n, out_d), jnp.float32))


def kernel(x, edge_index, W1, b1, W2, b2):
    n, d_in = x.shape
    d_h = W1.shape[0]
    d_out = W2.shape[0]
    e = edge_index.shape[1]

    src = edge_index[0].astype(jnp.int32)
    dst = edge_index[1].astype(jnp.int32)
    per_tile = _round_up(e, _NW * _CHUNK) // _NW
    n_chunks = per_tile // _CHUNK
    e_pad = per_tile * _NW
    if e_pad > e:
        src = jnp.concatenate([src, jnp.zeros((e_pad - e,), jnp.int32)])
        dst = jnp.concatenate([dst, jnp.full((e_pad - e,), n, jnp.int32)])
    srcr = src.reshape(_NC, _NS, n_chunks, _CHUNK)
    dstr = dst.reshape(_NC, _NS, n_chunks, _CHUNK)

    n_pad = _round_up(n + 1, _NS * _LANES)
    z2 = jnp.zeros((n_pad, d_h), jnp.float32)

    bn = 2000
    mm1 = pl.pallas_call(
        _matmul_body, grid=(n // bn,),
        in_specs=[pl.BlockSpec((bn, d_in), lambda i: (i, 0)),
                  pl.BlockSpec((d_in, d_h), lambda i: (0, 0))],
        out_specs=pl.BlockSpec((bn, d_h), lambda i: (i, 0)),
        out_shape=jax.ShapeDtypeStruct((n, d_h), jnp.float32))
    g1 = mm1(x, W1.T)

    scat1 = _make_sc_scatter(n_pad, n_chunks, d_h, with_count=True)
    parts1, counts = scat1(g1, srcr, dstr, z2)
    counts = counts.reshape(_NC, n_pad)

    p0 = parts1[0, :n]
    p1 = parts1[1, :n]
    c0 = counts[0, :n, None]
    c1 = counts[1, :n, None]

    mid = _tc_call(_mid_body, n, d_h, bn,
                   w_shape=(d_h, d_out), out_d=d_out)
    g2 = mid(p0, p1, c0, c1, b1.reshape(1, d_h), W2.T)

    scat2 = _make_sc_scatter(n_pad, n_chunks, d_out, with_count=False)
    (parts2,) = scat2(g2, srcr, dstr, z2)

    q0 = parts2[0, :n]
    q1 = parts2[1, :n]
    fin = _tc_call(_fin_body, n, d_out, bn)
    out = fin(q0, q1, c0, c1, b2.reshape(1, d_out))
    return out
